# baseline (speedup 1.0000x reference)
import jax
import jax.numpy as jnp
import numpy as np
from jax import lax
from jax.experimental import pallas as pl
from jax.experimental.pallas import tpu as pltpu
from jax.experimental.pallas import tpu_sc as plsc

K = 2048
R = 6
LANES = 16

_GUMBEL = np.array([
    float.fromhex("0x1.561c940000000p-2"),
    float.fromhex("0x1.e76f180000000p-1"),
    float.fromhex("0x1.7378be0000000p-1"),
    float.fromhex("0x1.18a9f00000000p-1"),
    float.fromhex("0x1.d07f1e0000000p-3"),
    float.fromhex("0x1.4092440000000p-1"),
], dtype=np.float32)


def _perm(v, idx):
    return v.at[idx].get(mode="promise_in_bounds")


def _lane_sum(v, iota):
    for sh in (8, 4, 2, 1):
        v = v + _perm(v, iota ^ sh)
    return v


def _lane_max(v, iota):
    for sh in (8, 4, 2, 1):
        v = jnp.maximum(v, _perm(v, iota ^ sh))
    return v


def _lane_min(v, iota):
    for sh in (8, 4, 2, 1):
        v = jnp.minimum(v, _perm(v, iota ^ sh))
    return v


def _make_router(g):
    mesh = plsc.VectorSubcoreMesh(
        core_axis_name="c", subcore_axis_name="s", num_cores=2,
        num_subcores=16)
    g_list = [float(v) for v in g]

    def body(x_hbm, w_hbm, b_hbm, out_hbm, x_v, w_v, b_v, out_v):
        c = lax.axis_index("c")
        s = lax.axis_index("s")
        iota = lax.iota(jnp.int32, LANES)

        @pl.when(jnp.logical_and(c == 0, s == 0))
        def _all():
            pltpu.sync_copy(x_hbm, x_v)
            pltpu.sync_copy(w_hbm, w_v)
            pltpu.sync_copy(b_hbm, b_v.at[pl.ds(0, R)])
            score = jnp.full((LANES,), -jnp.inf, jnp.float32)
            for row in range(R):
                acc = jnp.zeros((LANES,), jnp.float32)
                for i in range(K // LANES):
                    acc = acc + (w_v[pl.ds(row * K + i * LANES, LANES)]
                                 * x_v[pl.ds(i * LANES, LANES)])
                dot = _lane_sum(acc, iota)
                score = jnp.where(iota == row, dot, score)
            bsel = jnp.where(iota < R, b_v[...], jnp.float32(0.0))
            gv = jnp.full((LANES,), jnp.float32(0.0), jnp.float32)
            for t in range(R):
                gv = jnp.where(iota == t, jnp.float32(g_list[t]), gv)
            score = score + bsel + gv
            mx = _lane_max(score, iota)
            out_v[...] = _lane_min(
                jnp.where(score == mx, iota, jnp.int32(LANES)), iota)
            pltpu.sync_copy(out_v.at[pl.ds(0, 1)], out_hbm)

    return pl.kernel(
        body,
        out_type=jax.ShapeDtypeStruct((1,), jnp.int32),
        mesh=mesh,
        scratch_types=[
            pltpu.VMEM((K,), jnp.float32),
            pltpu.VMEM((R * K,), jnp.float32),
            pltpu.VMEM((LANES,), jnp.float32),
            pltpu.VMEM((LANES,), jnp.int32),
        ],
    )


def kernel(X, W, b):
    x = jnp.reshape(X, (-1,))
    w = jnp.reshape(W, (-1,))
    idx = _make_router(_GUMBEL)(x, w, b)
    return idx.astype(jnp.int64)


# speedup vs baseline: 5.8666x; 5.8666x over previous
import jax
import jax.numpy as jnp
import numpy as np
from jax import lax
from jax.experimental import pallas as pl
from jax.experimental.pallas import tpu as pltpu

K = 2048
R = 6

_GUMBEL = np.array([
    float.fromhex("0x1.561c940000000p-2"),
    float.fromhex("0x1.e76f180000000p-1"),
    float.fromhex("0x1.7378be0000000p-1"),
    float.fromhex("0x1.18a9f00000000p-1"),
    float.fromhex("0x1.d07f1e0000000p-3"),
    float.fromhex("0x1.4092440000000p-1"),
], dtype=np.float32)


def _body(x_ref, w_ref, b_ref, o_ref):
    xr = x_ref[...]
    best = jnp.float32(-jnp.inf)
    bidx = jnp.int32(0)
    for row in range(R):
        s_row = jnp.sum(w_ref[row] * xr) + b_ref[row] + jnp.float32(
            _GUMBEL[row])
        take = s_row > best
        bidx = jnp.where(take, jnp.int32(row), bidx)
        best = jnp.where(take, s_row, best)
    o_ref[0] = bidx


_ROUTER = pl.pallas_call(
    _body,
    out_shape=jax.ShapeDtypeStruct((1,), jnp.int32),
    in_specs=[
        pl.BlockSpec(memory_space=pltpu.VMEM),
        pl.BlockSpec(memory_space=pltpu.VMEM),
        pl.BlockSpec(memory_space=pltpu.SMEM),
    ],
    out_specs=pl.BlockSpec(memory_space=pltpu.SMEM),
)


def kernel(X, W, b):
    x2 = jnp.reshape(X, (16, 128))
    w3 = jnp.reshape(W, (R, 16, 128))
    idx = _ROUTER(x2, w3, b)
    return idx.astype(jnp.int64)
